# baseline (speedup 1.0000x reference)
import functools

import jax
import jax.numpy as jnp
from jax import lax
from jax.experimental import pallas as pl
from jax.experimental.pallas import tpu as pltpu
from jax.experimental.pallas import tpu_sc as plsc

_NC = 2
_NS = 16
_NW = _NC * _NS

_C = 128
_NPAD = 10240


def _node_mlp(x, W1, b1, W2, b2, interpret=False):
    N = x.shape[0]
    BN = 1000

    def body(x_ref, W1_ref, b1_ref, W2_ref, b2_ref, o_ref):
        h = jnp.maximum(
            jnp.dot(x_ref[...], W1_ref[...], preferred_element_type=jnp.float32)
            + b1_ref[...], 0.0)
        o_ref[...] = (
            jnp.dot(h, W2_ref[...], preferred_element_type=jnp.float32)
            + b2_ref[...])

    return pl.pallas_call(
        body,
        grid=(N // BN,),
        in_specs=[
            pl.BlockSpec((BN, 128), lambda i: (i, 0)),
            pl.BlockSpec((128, 128), lambda i: (0, 0)),
            pl.BlockSpec((1, 128), lambda i: (0, 0)),
            pl.BlockSpec((128, 128), lambda i: (0, 0)),
            pl.BlockSpec((1, 128), lambda i: (0, 0)),
        ],
        out_specs=pl.BlockSpec((BN, 128), lambda i: (i, 0)),
        out_shape=jax.ShapeDtypeStruct((N, 128), jnp.float32),
        interpret=interpret,
    )(x, W1, b1.reshape(1, 128), W2, b2.reshape(1, 128))


def _sc_gather(table, idx):
    D = table.shape[1]
    E = idx.shape[0]
    per_w = E // _NW
    full = per_w // _C
    rem = per_w - full * _C
    NBUF = 3
    LOOK = 2
    ngrp = full // NBUF
    assert ngrp * NBUF == full and per_w % 8 == 0
    mesh = plsc.VectorSubcoreMesh(core_axis_name="c", subcore_axis_name="s")

    @functools.partial(
        pl.kernel,
        out_type=jax.ShapeDtypeStruct((E, D), jnp.float32),
        mesh=mesh,
        scratch_types=(
            [pltpu.VMEM((per_w,), jnp.int32),
             pltpu.VMEM((NBUF, _C, D), jnp.float32),
             pltpu.VMEM((rem,), jnp.int32),
             pltpu.VMEM((rem, D), jnp.float32)]
            + [pltpu.SemaphoreType.DMA] * (2 * NBUF)
        ),
    )
    def k(table_hbm, idx_hbm, out_hbm, idx_v, rows_v, idx_r, rows_r, *sems):
        gsem = sems[:NBUF]
        wsem = sems[NBUF:]
        wid = lax.axis_index("s") * _NC + lax.axis_index("c")
        base = wid * per_w
        pltpu.sync_copy(idx_hbm.at[pl.ds(base, per_w)], idx_v)
        for b in range(LOOK):
            pltpu.async_copy(
                table_hbm.at[idx_v.at[pl.ds(b * _C, _C)]],
                rows_v.at[b], gsem[b])

        def outer(g, _):
            for b in range(NBUF):
                c = g * NBUF + b
                b2 = (b + LOOK) % NBUF

                @pl.when(c + LOOK - NBUF >= 0)
                def _():
                    pltpu.make_async_copy(
                        rows_v.at[b2], out_hbm.at[pl.ds(0, _C)],
                        wsem[b2]).wait()

                @pl.when(c + LOOK < full)
                def _():
                    pltpu.async_copy(
                        table_hbm.at[idx_v.at[pl.ds((c + LOOK) * _C, _C)]],
                        rows_v.at[b2], gsem[b2])

                pltpu.make_async_copy(
                    table_hbm.at[pl.ds(0, _C)], rows_v.at[b],
                    gsem[b]).wait()
                pltpu.async_copy(rows_v.at[b],
                                 out_hbm.at[pl.ds(base + c * _C, _C)],
                                 wsem[b])
            return ()

        lax.fori_loop(0, ngrp, outer, ())
        pltpu.make_async_copy(rows_v.at[(full - 1) % NBUF],
                              out_hbm.at[pl.ds(0, _C)],
                              wsem[(full - 1) % NBUF]).wait()
        off = base + full * _C
        pltpu.sync_copy(idx_hbm.at[pl.ds(off, rem)], idx_r)
        pltpu.async_copy(table_hbm.at[idx_r], rows_r, gsem[0]).wait()
        pltpu.sync_copy(rows_r, out_hbm.at[pl.ds(off, rem)])

    return k(table, idx)


def _sc_gather_pair(table, idx_a, idx_b):
    D = table.shape[1]
    E = idx_a.shape[0]
    per_w = E // _NW
    full = per_w // _C
    rem = per_w - full * _C
    NBUF = 3
    LOOK = 2
    ngrp = full // NBUF
    assert ngrp * NBUF == full and per_w % 8 == 0
    mesh = plsc.VectorSubcoreMesh(core_axis_name="c", subcore_axis_name="s")

    @functools.partial(
        pl.kernel,
        out_type=(jax.ShapeDtypeStruct((E, D), jnp.float32),
                  jax.ShapeDtypeStruct((E, D), jnp.float32)),
        mesh=mesh,
        scratch_types=(
            [pltpu.VMEM((per_w,), jnp.int32),
             pltpu.VMEM((per_w,), jnp.int32),
             pltpu.VMEM((NBUF, _C, D), jnp.float32),
             pltpu.VMEM((NBUF, _C, D), jnp.float32),
             pltpu.VMEM((rem,), jnp.int32),
             pltpu.VMEM((rem, D), jnp.float32)]
            + [pltpu.SemaphoreType.DMA] * (4 * NBUF)
        ),
    )
    def k(table_hbm, ia_hbm, ib_hbm, oa_hbm, ob_hbm, ia_v, ib_v, ra_v, rb_v,
          idx_r, rows_r, *sems):
        gsa = sems[:NBUF]
        gsb = sems[NBUF:2 * NBUF]
        wsa = sems[2 * NBUF:3 * NBUF]
        wsb = sems[3 * NBUF:]
        wid = lax.axis_index("s") * _NC + lax.axis_index("c")
        base = wid * per_w
        pltpu.sync_copy(ia_hbm.at[pl.ds(base, per_w)], ia_v)
        pltpu.sync_copy(ib_hbm.at[pl.ds(base, per_w)], ib_v)
        for b in range(LOOK):
            pltpu.async_copy(table_hbm.at[ia_v.at[pl.ds(b * _C, _C)]],
                             ra_v.at[b], gsa[b])
            pltpu.async_copy(table_hbm.at[ib_v.at[pl.ds(b * _C, _C)]],
                             rb_v.at[b], gsb[b])

        def outer(g, _):
            for b in range(NBUF):
                c = g * NBUF + b
                b2 = (b + LOOK) % NBUF
                for (rv, gs, ws, oh, iv) in ((ra_v, gsa, wsa, oa_hbm, ia_v),
                                             (rb_v, gsb, wsb, ob_hbm, ib_v)):

                    @pl.when(c + LOOK - NBUF >= 0)
                    def _():
                        pltpu.make_async_copy(
                            rv.at[b2], oh.at[pl.ds(0, _C)], ws[b2]).wait()

                    @pl.when(c + LOOK < full)
                    def _():
                        pltpu.async_copy(
                            table_hbm.at[iv.at[pl.ds((c + LOOK) * _C, _C)]],
                            rv.at[b2], gs[b2])

                    pltpu.make_async_copy(
                        table_hbm.at[pl.ds(0, _C)], rv.at[b], gs[b]).wait()
                    pltpu.async_copy(rv.at[b],
                                     oh.at[pl.ds(base + c * _C, _C)], ws[b])
            return ()

        lax.fori_loop(0, ngrp, outer, ())
        for (rv, ws, oh) in ((ra_v, wsa, oa_hbm), (rb_v, wsb, ob_hbm)):
            pltpu.make_async_copy(rv.at[(full - 1) % NBUF],
                                  oh.at[pl.ds(0, _C)],
                                  ws[(full - 1) % NBUF]).wait()
        off = base + full * _C
        for (ih, oh) in ((ia_hbm, oa_hbm), (ib_hbm, ob_hbm)):
            pltpu.sync_copy(ih.at[pl.ds(off, rem)], idx_r)
            pltpu.async_copy(table_hbm.at[idx_r], rows_r, gsa[0]).wait()
            pltpu.sync_copy(rows_r, oh.at[pl.ds(off, rem)])

    return k(table, idx_a, idx_b)


def _sc_scatter_add(m, dst, zeros):
    E = m.shape[0]
    per_w = E // _NW
    full = per_w // _C
    rem = per_w - full * _C
    NBUF = 2
    ngrp = full // NBUF
    tail = full - ngrp * NBUF
    mesh = plsc.VectorSubcoreMesh(core_axis_name="c", subcore_axis_name="s")
    rpt = _NPAD // _NS

    @functools.partial(
        pl.kernel,
        out_type=jax.ShapeDtypeStruct((_NC, _NPAD, 128), jnp.float32),
        mesh=mesh,
        scratch_types=(
            [pltpu.VMEM((NBUF, _C), jnp.int32),
             pltpu.VMEM((NBUF, _C, 128), jnp.float32),
             pltpu.VMEM((rem,), jnp.int32),
             pltpu.VMEM((rem, 128), jnp.float32),
             pltpu.VMEM_SHARED((_NPAD, 128), jnp.float32)]
            + [pltpu.SemaphoreType.DMA] * (3 * NBUF)
        ),
    )
    def k(m_hbm, dst_hbm, z_hbm, out_hbm, idx_v, rows_v, idx_r, rows_r,
          agg_sh, *sems):
        isem = sems[:NBUF]
        lsem = sems[NBUF:2 * NBUF]
        ssem = sems[2 * NBUF:]
        cid = lax.axis_index("c")
        sid = lax.axis_index("s")
        wid = sid * _NC + cid
        base = wid * per_w
        pltpu.sync_copy(z_hbm, agg_sh.at[pl.ds(sid * rpt, rpt)])
        plsc.subcore_barrier()
        for b in range(NBUF):
            off = base + b * _C
            pltpu.async_copy(dst_hbm.at[pl.ds(off, _C)], idx_v.at[b],
                             isem[b])
            pltpu.async_copy(m_hbm.at[pl.ds(off, _C)], rows_v.at[b],
                             lsem[b])

        def outer(g, _):
            for b in range(NBUF):
                c = g * NBUF + b
                pltpu.make_async_copy(dst_hbm.at[pl.ds(0, _C)],
                                      idx_v.at[b], isem[b]).wait()
                pltpu.make_async_copy(m_hbm.at[pl.ds(0, _C)],
                                      rows_v.at[b], lsem[b]).wait()
                pltpu.async_copy(rows_v.at[b], agg_sh.at[idx_v.at[b]],
                                 ssem[b], add=True)
                pltpu.make_async_copy(rows_v.at[b], agg_sh.at[idx_v.at[b]],
                                      ssem[b]).wait()

                @pl.when(c + NBUF < full)
                def _():
                    off = base + (c + NBUF) * _C
                    pltpu.async_copy(dst_hbm.at[pl.ds(off, _C)],
                                     idx_v.at[b], isem[b])
                    pltpu.async_copy(m_hbm.at[pl.ds(off, _C)],
                                     rows_v.at[b], lsem[b])
            return ()

        lax.fori_loop(0, ngrp, outer, ())
        for t in range(tail):
            c = ngrp * NBUF + t
            b = c % NBUF
            pltpu.make_async_copy(dst_hbm.at[pl.ds(0, _C)],
                                  idx_v.at[b], isem[b]).wait()
            pltpu.make_async_copy(m_hbm.at[pl.ds(0, _C)],
                                  rows_v.at[b], lsem[b]).wait()
            pltpu.sync_copy(rows_v.at[b], agg_sh.at[idx_v.at[b]], add=True)
        off = base + full * _C
        pltpu.sync_copy(dst_hbm.at[pl.ds(off, rem)], idx_r)
        pltpu.sync_copy(m_hbm.at[pl.ds(off, rem)], rows_r)
        pltpu.sync_copy(rows_r, agg_sh.at[idx_r], add=True)
        plsc.subcore_barrier()
        pltpu.sync_copy(agg_sh.at[pl.ds(sid * rpt, rpt)],
                        out_hbm.at[cid, pl.ds(sid * rpt, rpt)])

    return k(m, dst, zeros)


def _edge_msg(g1, ea, We1, be1, We2, be2, Wc, bc, interpret=False):
    E = g1.shape[0]
    BE = 4000

    def body(g1_ref, ea_ref, We1_ref, be1_ref, We2_ref, be2_ref, Wc_ref,
             bc_ref, o_ref):
        h = jnp.maximum(
            jnp.dot(ea_ref[...], We1_ref[...], preferred_element_type=jnp.float32)
            + be1_ref[...], 0.0)
        m2 = (jnp.dot(h, We2_ref[...], preferred_element_type=jnp.float32)
              + be2_ref[...])
        t = g1_ref[...] * m2
        o_ref[...] = jnp.tanh(
            jnp.dot(t, Wc_ref[...], preferred_element_type=jnp.float32)
            + bc_ref[...])

    return pl.pallas_call(
        body,
        grid=(E // BE,),
        in_specs=[
            pl.BlockSpec((BE, 128), lambda i: (i, 0)),
            pl.BlockSpec((BE, 16), lambda i: (i, 0)),
            pl.BlockSpec((16, 128), lambda i: (0, 0)),
            pl.BlockSpec((1, 128), lambda i: (0, 0)),
            pl.BlockSpec((128, 128), lambda i: (0, 0)),
            pl.BlockSpec((1, 128), lambda i: (0, 0)),
            pl.BlockSpec((128, 128), lambda i: (0, 0)),
            pl.BlockSpec((1, 128), lambda i: (0, 0)),
        ],
        out_specs=pl.BlockSpec((BE, 128), lambda i: (i, 0)),
        out_shape=jax.ShapeDtypeStruct((E, 128), jnp.float32),
        interpret=interpret,
    )(g1, ea, We1, be1.reshape(1, 128), We2, be2.reshape(1, 128), Wc,
      bc.reshape(1, 128))


def _combine(p00, p01, p10, p11, x, interpret=False):
    N = x.shape[0]
    BN = 1000

    def body(a_ref, b_ref, c_ref, d_ref, x_ref, o_ref):
        o_ref[...] = (a_ref[...] + b_ref[...]) + (c_ref[...] + d_ref[...]) \
            + x_ref[...]

    return pl.pallas_call(
        body,
        grid=(N // BN,),
        in_specs=[pl.BlockSpec((BN, 128), lambda i: (i, 0))] * 5,
        out_specs=pl.BlockSpec((BN, 128), lambda i: (i, 0)),
        out_shape=jax.ShapeDtypeStruct((N, 128), jnp.float32),
        interpret=interpret,
    )(p00, p01, p10, p11, x)


def _edge_update(hs, hd, ea, Wu, interpret=False):
    E = hs.shape[0]
    BE = 4000

    def body(hs_ref, hd_ref, ea_ref, Wu_ref, o_ref):
        prod = hs_ref[...] * hd_ref[...]
        o_ref[...] = 0.8 * ea_ref[...] + 0.2 * jnp.dot(
            prod, Wu_ref[...], preferred_element_type=jnp.float32)

    return pl.pallas_call(
        body,
        grid=(E // BE,),
        in_specs=[
            pl.BlockSpec((BE, 128), lambda i: (i, 0)),
            pl.BlockSpec((BE, 128), lambda i: (i, 0)),
            pl.BlockSpec((BE, 16), lambda i: (i, 0)),
            pl.BlockSpec((128, 16), lambda i: (0, 0)),
        ],
        out_specs=pl.BlockSpec((BE, 16), lambda i: (i, 0)),
        out_shape=jax.ShapeDtypeStruct((E, 16), jnp.float32),
        interpret=interpret,
    )(hs, hd, ea, Wu)


def kernel(x, edge_index, edge_attr, W1, b1, W2, b2, We1, be1, We2, be2,
           Wc, bc, Wu):
    E = edge_index.shape[1]
    EH = E // 2
    src = edge_index[0]
    dst = edge_index[1]
    src0, src1 = src[:EH], src[EH:]
    dst0, dst1 = dst[:EH], dst[EH:]
    ea0, ea1 = edge_attr[:EH], edge_attr[EH:]

    node_m1 = _node_mlp(x, W1, b1, W2, b2)
    g1_0 = _sc_gather(node_m1, src0)
    g1_1 = _sc_gather(node_m1, src1)
    m0 = _edge_msg(g1_0, ea0, We1, be1, We2, be2, Wc, bc)
    m1 = _edge_msg(g1_1, ea1, We1, be1, We2, be2, Wc, bc)
    zeros = jnp.zeros((_NPAD // _NS, 128), jnp.float32)
    p0 = _sc_scatter_add(m0, dst0, zeros)
    p1 = _sc_scatter_add(m1, dst1, zeros)
    h_new = _combine(p0[0, :10000], p0[1, :10000], p1[0, :10000],
                     p1[1, :10000], x)
    hs0, hd0 = _sc_gather_pair(h_new, src0, dst0)
    hs1, hd1 = _sc_gather_pair(h_new, src1, dst1)
    e0 = _edge_update(hs0, hd0, ea0, Wu)
    e1 = _edge_update(hs1, hd1, ea1, Wu)
    e_new = jnp.concatenate([e0, e1], axis=0)
    return (h_new, e_new)


# speedup vs baseline: 1.1329x; 1.1329x over previous
import functools

import jax
import jax.numpy as jnp
from jax import lax
from jax.experimental import pallas as pl
from jax.experimental.pallas import tpu as pltpu
from jax.experimental.pallas import tpu_sc as plsc

_NC = 2
_NS = 16
_NW = _NC * _NS

_C = 128
_NPAD = 10240


def _node_mlp(x, W1, b1, W2, b2, interpret=False):
    N = x.shape[0]
    BN = 1000

    def body(x_ref, W1_ref, b1_ref, W2_ref, b2_ref, o_ref):
        h = jnp.maximum(
            jnp.dot(x_ref[...], W1_ref[...], preferred_element_type=jnp.float32)
            + b1_ref[...], 0.0)
        o_ref[...] = (
            jnp.dot(h, W2_ref[...], preferred_element_type=jnp.float32)
            + b2_ref[...])

    return pl.pallas_call(
        body,
        grid=(N // BN,),
        in_specs=[
            pl.BlockSpec((BN, 128), lambda i: (i, 0)),
            pl.BlockSpec((128, 128), lambda i: (0, 0)),
            pl.BlockSpec((1, 128), lambda i: (0, 0)),
            pl.BlockSpec((128, 128), lambda i: (0, 0)),
            pl.BlockSpec((1, 128), lambda i: (0, 0)),
        ],
        out_specs=pl.BlockSpec((BN, 128), lambda i: (i, 0)),
        out_shape=jax.ShapeDtypeStruct((N, 128), jnp.float32),
        interpret=interpret,
    )(x, W1, b1.reshape(1, 128), W2, b2.reshape(1, 128))


def _sc_gather(table, idx):
    D = table.shape[1]
    E = idx.shape[0]
    per_w = E // _NW
    full = per_w // _C
    rem = per_w - full * _C
    NBUF = 3
    LOOK = 2
    ngrp = full // NBUF
    assert ngrp * NBUF == full and per_w % 8 == 0
    mesh = plsc.VectorSubcoreMesh(core_axis_name="c", subcore_axis_name="s")

    @functools.partial(
        pl.kernel,
        out_type=jax.ShapeDtypeStruct((E, D), jnp.float32),
        mesh=mesh,
        scratch_types=(
            [pltpu.VMEM((per_w,), jnp.int32),
             pltpu.VMEM((NBUF, _C, D), jnp.float32),
             pltpu.VMEM((rem,), jnp.int32),
             pltpu.VMEM((rem, D), jnp.float32)]
            + [pltpu.SemaphoreType.DMA] * (2 * NBUF)
        ),
    )
    def k(table_hbm, idx_hbm, out_hbm, idx_v, rows_v, idx_r, rows_r, *sems):
        gsem = sems[:NBUF]
        wsem = sems[NBUF:]
        wid = lax.axis_index("s") * _NC + lax.axis_index("c")
        base = wid * per_w
        pltpu.sync_copy(idx_hbm.at[pl.ds(base, per_w)], idx_v)
        for b in range(LOOK):
            pltpu.async_copy(
                table_hbm.at[idx_v.at[pl.ds(b * _C, _C)]],
                rows_v.at[b], gsem[b])

        def outer(g, _):
            for b in range(NBUF):
                c = g * NBUF + b
                b2 = (b + LOOK) % NBUF

                @pl.when(c + LOOK - NBUF >= 0)
                def _():
                    pltpu.make_async_copy(
                        rows_v.at[b2], out_hbm.at[pl.ds(0, _C)],
                        wsem[b2]).wait()

                @pl.when(c + LOOK < full)
                def _():
                    pltpu.async_copy(
                        table_hbm.at[idx_v.at[pl.ds((c + LOOK) * _C, _C)]],
                        rows_v.at[b2], gsem[b2])

                pltpu.make_async_copy(
                    table_hbm.at[pl.ds(0, _C)], rows_v.at[b],
                    gsem[b]).wait()
                pltpu.async_copy(rows_v.at[b],
                                 out_hbm.at[pl.ds(base + c * _C, _C)],
                                 wsem[b])
            return ()

        lax.fori_loop(0, ngrp, outer, ())
        pltpu.make_async_copy(rows_v.at[(full - 1) % NBUF],
                              out_hbm.at[pl.ds(0, _C)],
                              wsem[(full - 1) % NBUF]).wait()
        off = base + full * _C
        pltpu.sync_copy(idx_hbm.at[pl.ds(off, rem)], idx_r)
        pltpu.async_copy(table_hbm.at[idx_r], rows_r, gsem[0]).wait()
        pltpu.sync_copy(rows_r, out_hbm.at[pl.ds(off, rem)])

    return k(table, idx)


def _sc_gather_pair(table, idx_a, idx_b):
    D = table.shape[1]
    E = idx_a.shape[0]
    per_w = E // _NW
    full = per_w // _C
    rem = per_w - full * _C
    NBUF = 3
    LOOK = 2
    ngrp = full // NBUF
    assert ngrp * NBUF == full and per_w % 8 == 0
    mesh = plsc.VectorSubcoreMesh(core_axis_name="c", subcore_axis_name="s")

    @functools.partial(
        pl.kernel,
        out_type=(jax.ShapeDtypeStruct((E, D), jnp.float32),
                  jax.ShapeDtypeStruct((E, D), jnp.float32)),
        mesh=mesh,
        scratch_types=(
            [pltpu.VMEM((per_w,), jnp.int32),
             pltpu.VMEM((per_w,), jnp.int32),
             pltpu.VMEM((NBUF, _C, D), jnp.float32),
             pltpu.VMEM((NBUF, _C, D), jnp.float32),
             pltpu.VMEM((rem,), jnp.int32),
             pltpu.VMEM((rem, D), jnp.float32)]
            + [pltpu.SemaphoreType.DMA] * (4 * NBUF)
        ),
    )
    def k(table_hbm, ia_hbm, ib_hbm, oa_hbm, ob_hbm, ia_v, ib_v, ra_v, rb_v,
          idx_r, rows_r, *sems):
        gsa = sems[:NBUF]
        gsb = sems[NBUF:2 * NBUF]
        wsa = sems[2 * NBUF:3 * NBUF]
        wsb = sems[3 * NBUF:]
        wid = lax.axis_index("s") * _NC + lax.axis_index("c")
        base = wid * per_w
        pltpu.sync_copy(ia_hbm.at[pl.ds(base, per_w)], ia_v)
        pltpu.sync_copy(ib_hbm.at[pl.ds(base, per_w)], ib_v)
        for b in range(LOOK):
            pltpu.async_copy(table_hbm.at[ia_v.at[pl.ds(b * _C, _C)]],
                             ra_v.at[b], gsa[b])
            pltpu.async_copy(table_hbm.at[ib_v.at[pl.ds(b * _C, _C)]],
                             rb_v.at[b], gsb[b])

        def outer(g, _):
            for b in range(NBUF):
                c = g * NBUF + b
                b2 = (b + LOOK) % NBUF
                for (rv, gs, ws, oh, iv) in ((ra_v, gsa, wsa, oa_hbm, ia_v),
                                             (rb_v, gsb, wsb, ob_hbm, ib_v)):

                    @pl.when(c + LOOK - NBUF >= 0)
                    def _():
                        pltpu.make_async_copy(
                            rv.at[b2], oh.at[pl.ds(0, _C)], ws[b2]).wait()

                    @pl.when(c + LOOK < full)
                    def _():
                        pltpu.async_copy(
                            table_hbm.at[iv.at[pl.ds((c + LOOK) * _C, _C)]],
                            rv.at[b2], gs[b2])

                    pltpu.make_async_copy(
                        table_hbm.at[pl.ds(0, _C)], rv.at[b], gs[b]).wait()
                    pltpu.async_copy(rv.at[b],
                                     oh.at[pl.ds(base + c * _C, _C)], ws[b])
            return ()

        lax.fori_loop(0, ngrp, outer, ())
        for (rv, ws, oh) in ((ra_v, wsa, oa_hbm), (rb_v, wsb, ob_hbm)):
            pltpu.make_async_copy(rv.at[(full - 1) % NBUF],
                                  oh.at[pl.ds(0, _C)],
                                  ws[(full - 1) % NBUF]).wait()
        off = base + full * _C
        for (ih, oh) in ((ia_hbm, oa_hbm), (ib_hbm, ob_hbm)):
            pltpu.sync_copy(ih.at[pl.ds(off, rem)], idx_r)
            pltpu.async_copy(table_hbm.at[idx_r], rows_r, gsa[0]).wait()
            pltpu.sync_copy(rows_r, oh.at[pl.ds(off, rem)])

    return k(table, idx_a, idx_b)


def _sc_gather_prod(table, idx_a, idx_b):
    D = table.shape[1]
    E = idx_a.shape[0]
    per_w = E // _NW
    full = per_w // _C
    rem = per_w - full * _C
    NBUF = 3
    LOOK = 2
    ngrp = full // NBUF
    assert ngrp * NBUF == full and per_w % 8 == 0 and D % 16 == 0
    mesh = plsc.VectorSubcoreMesh(core_axis_name="c", subcore_axis_name="s")

    @functools.partial(
        pl.kernel,
        out_type=jax.ShapeDtypeStruct((E, D), jnp.float32),
        mesh=mesh,
        scratch_types=(
            [pltpu.VMEM((per_w,), jnp.int32),
             pltpu.VMEM((per_w,), jnp.int32),
             pltpu.VMEM((NBUF, _C, D), jnp.float32),
             pltpu.VMEM((NBUF, _C, D), jnp.float32),
             pltpu.VMEM((rem,), jnp.int32),
             pltpu.VMEM((rem, D), jnp.float32),
             pltpu.VMEM((rem, D), jnp.float32)]
            + [pltpu.SemaphoreType.DMA] * (3 * NBUF)
        ),
    )
    def k(table_hbm, ia_hbm, ib_hbm, o_hbm, ia_v, ib_v, ra_v, rb_v,
          idx_r, rowsa_r, rowsb_r, *sems):
        gsa = sems[:NBUF]
        gsb = sems[NBUF:2 * NBUF]
        wsa = sems[2 * NBUF:]
        wid = lax.axis_index("s") * _NC + lax.axis_index("c")
        base = wid * per_w
        pltpu.sync_copy(ia_hbm.at[pl.ds(base, per_w)], ia_v)
        pltpu.sync_copy(ib_hbm.at[pl.ds(base, per_w)], ib_v)
        for b in range(LOOK):
            pltpu.async_copy(table_hbm.at[ia_v.at[pl.ds(b * _C, _C)]],
                             ra_v.at[b], gsa[b])
            pltpu.async_copy(table_hbm.at[ib_v.at[pl.ds(b * _C, _C)]],
                             rb_v.at[b], gsb[b])

        def outer(g, _):
            for b in range(NBUF):
                c = g * NBUF + b
                b2 = (b + LOOK) % NBUF

                @pl.when(c + LOOK - NBUF >= 0)
                def _():
                    pltpu.make_async_copy(
                        ra_v.at[b2], o_hbm.at[pl.ds(0, _C)], wsa[b2]).wait()

                @pl.when(c + LOOK < full)
                def _():
                    pltpu.async_copy(
                        table_hbm.at[ia_v.at[pl.ds((c + LOOK) * _C, _C)]],
                        ra_v.at[b2], gsa[b2])
                    pltpu.async_copy(
                        table_hbm.at[ib_v.at[pl.ds((c + LOOK) * _C, _C)]],
                        rb_v.at[b2], gsb[b2])

                pltpu.make_async_copy(
                    table_hbm.at[pl.ds(0, _C)], ra_v.at[b], gsa[b]).wait()
                pltpu.make_async_copy(
                    table_hbm.at[pl.ds(0, _C)], rb_v.at[b], gsb[b]).wait()

                def mul_row(i, _):
                    for j in range(D // 16):
                        ra_v[b, i, pl.ds(16 * j, 16)] = (
                            ra_v[b, i, pl.ds(16 * j, 16)]
                            * rb_v[b, i, pl.ds(16 * j, 16)])
                    return ()

                lax.fori_loop(0, _C, mul_row, ())
                pltpu.async_copy(ra_v.at[b],
                                 o_hbm.at[pl.ds(base + c * _C, _C)], wsa[b])
            return ()

        lax.fori_loop(0, ngrp, outer, ())
        pltpu.make_async_copy(ra_v.at[(full - 1) % NBUF],
                              o_hbm.at[pl.ds(0, _C)],
                              wsa[(full - 1) % NBUF]).wait()
        off = base + full * _C
        pltpu.sync_copy(ia_hbm.at[pl.ds(off, rem)], idx_r)
        pltpu.async_copy(table_hbm.at[idx_r], rowsa_r, gsa[0]).wait()
        pltpu.sync_copy(ib_hbm.at[pl.ds(off, rem)], idx_r)
        pltpu.async_copy(table_hbm.at[idx_r], rowsb_r, gsb[0]).wait()
        for i in range(rem):
            for j in range(D // 16):
                rowsa_r[i, pl.ds(16 * j, 16)] = (
                    rowsa_r[i, pl.ds(16 * j, 16)]
                    * rowsb_r[i, pl.ds(16 * j, 16)])
        pltpu.sync_copy(rowsa_r, o_hbm.at[pl.ds(off, rem)])

    return k(table, idx_a, idx_b)


def _sc_scatter_add(m, dst, zeros):
    E = m.shape[0]
    per_w = E // _NW
    full = per_w // _C
    rem = per_w - full * _C
    NBUF = 2
    ngrp = full // NBUF
    tail = full - ngrp * NBUF
    mesh = plsc.VectorSubcoreMesh(core_axis_name="c", subcore_axis_name="s")
    rpt = _NPAD // _NS

    @functools.partial(
        pl.kernel,
        out_type=jax.ShapeDtypeStruct((_NC, _NPAD, 128), jnp.float32),
        mesh=mesh,
        scratch_types=(
            [pltpu.VMEM((NBUF, _C), jnp.int32),
             pltpu.VMEM((NBUF, _C, 128), jnp.float32),
             pltpu.VMEM((rem,), jnp.int32),
             pltpu.VMEM((rem, 128), jnp.float32),
             pltpu.VMEM_SHARED((_NPAD, 128), jnp.float32)]
            + [pltpu.SemaphoreType.DMA] * (3 * NBUF)
        ),
    )
    def k(m_hbm, dst_hbm, z_hbm, out_hbm, idx_v, rows_v, idx_r, rows_r,
          agg_sh, *sems):
        isem = sems[:NBUF]
        lsem = sems[NBUF:2 * NBUF]
        ssem = sems[2 * NBUF:]
        cid = lax.axis_index("c")
        sid = lax.axis_index("s")
        wid = sid * _NC + cid
        base = wid * per_w
        pltpu.sync_copy(z_hbm, agg_sh.at[pl.ds(sid * rpt, rpt)])
        plsc.subcore_barrier()
        for b in range(NBUF):
            off = base + b * _C
            pltpu.async_copy(dst_hbm.at[pl.ds(off, _C)], idx_v.at[b],
                             isem[b])
            pltpu.async_copy(m_hbm.at[pl.ds(off, _C)], rows_v.at[b],
                             lsem[b])

        def outer(g, _):
            for b in range(NBUF):
                c = g * NBUF + b
                pltpu.make_async_copy(dst_hbm.at[pl.ds(0, _C)],
                                      idx_v.at[b], isem[b]).wait()
                pltpu.make_async_copy(m_hbm.at[pl.ds(0, _C)],
                                      rows_v.at[b], lsem[b]).wait()
                pltpu.async_copy(rows_v.at[b], agg_sh.at[idx_v.at[b]],
                                 ssem[b], add=True)
                pltpu.make_async_copy(rows_v.at[b], agg_sh.at[idx_v.at[b]],
                                      ssem[b]).wait()

                @pl.when(c + NBUF < full)
                def _():
                    off = base + (c + NBUF) * _C
                    pltpu.async_copy(dst_hbm.at[pl.ds(off, _C)],
                                     idx_v.at[b], isem[b])
                    pltpu.async_copy(m_hbm.at[pl.ds(off, _C)],
                                     rows_v.at[b], lsem[b])
            return ()

        lax.fori_loop(0, ngrp, outer, ())
        for t in range(tail):
            c = ngrp * NBUF + t
            b = c % NBUF
            pltpu.make_async_copy(dst_hbm.at[pl.ds(0, _C)],
                                  idx_v.at[b], isem[b]).wait()
            pltpu.make_async_copy(m_hbm.at[pl.ds(0, _C)],
                                  rows_v.at[b], lsem[b]).wait()
            pltpu.sync_copy(rows_v.at[b], agg_sh.at[idx_v.at[b]], add=True)
        off = base + full * _C
        pltpu.sync_copy(dst_hbm.at[pl.ds(off, rem)], idx_r)
        pltpu.sync_copy(m_hbm.at[pl.ds(off, rem)], rows_r)
        pltpu.sync_copy(rows_r, agg_sh.at[idx_r], add=True)
        plsc.subcore_barrier()
        pltpu.sync_copy(agg_sh.at[pl.ds(sid * rpt, rpt)],
                        out_hbm.at[cid, pl.ds(sid * rpt, rpt)])

    return k(m, dst, zeros)


def _edge_msg(g1, ea, We1, be1, We2, be2, Wc, bc, interpret=False):
    E = g1.shape[0]
    BE = 4000

    def body(g1_ref, ea_ref, We1_ref, be1_ref, We2_ref, be2_ref, Wc_ref,
             bc_ref, o_ref):
        h = jnp.maximum(
            jnp.dot(ea_ref[...], We1_ref[...], preferred_element_type=jnp.float32)
            + be1_ref[...], 0.0)
        m2 = (jnp.dot(h, We2_ref[...], preferred_element_type=jnp.float32)
              + be2_ref[...])
        t = g1_ref[...] * m2
        o_ref[...] = jnp.tanh(
            jnp.dot(t, Wc_ref[...], preferred_element_type=jnp.float32)
            + bc_ref[...])

    return pl.pallas_call(
        body,
        grid=(E // BE,),
        in_specs=[
            pl.BlockSpec((BE, 128), lambda i: (i, 0)),
            pl.BlockSpec((BE, 16), lambda i: (i, 0)),
            pl.BlockSpec((16, 128), lambda i: (0, 0)),
            pl.BlockSpec((1, 128), lambda i: (0, 0)),
            pl.BlockSpec((128, 128), lambda i: (0, 0)),
            pl.BlockSpec((1, 128), lambda i: (0, 0)),
            pl.BlockSpec((128, 128), lambda i: (0, 0)),
            pl.BlockSpec((1, 128), lambda i: (0, 0)),
        ],
        out_specs=pl.BlockSpec((BE, 128), lambda i: (i, 0)),
        out_shape=jax.ShapeDtypeStruct((E, 128), jnp.float32),
        interpret=interpret,
    )(g1, ea, We1, be1.reshape(1, 128), We2, be2.reshape(1, 128), Wc,
      bc.reshape(1, 128))


def _combine(p0, p1, x, interpret=False):
    N = x.shape[0]
    BN = 1000

    def body(a_ref, b_ref, x_ref, o_ref):
        o_ref[...] = a_ref[...] + b_ref[...] + x_ref[...]

    return pl.pallas_call(
        body,
        grid=(N // BN,),
        in_specs=[pl.BlockSpec((BN, 128), lambda i: (i, 0))] * 3,
        out_specs=pl.BlockSpec((BN, 128), lambda i: (i, 0)),
        out_shape=jax.ShapeDtypeStruct((N, 128), jnp.float32),
        interpret=interpret,
    )(p0, p1, x)


def _edge_update(prod, ea, Wu, interpret=False):
    E = prod.shape[0]
    BE = 4000

    def body(pr_ref, ea_ref, Wu_ref, o_ref):
        o_ref[...] = 0.8 * ea_ref[...] + 0.2 * jnp.dot(
            pr_ref[...], Wu_ref[...], preferred_element_type=jnp.float32)

    return pl.pallas_call(
        body,
        grid=(E // BE,),
        in_specs=[
            pl.BlockSpec((BE, 128), lambda i: (i, 0)),
            pl.BlockSpec((BE, 16), lambda i: (i, 0)),
            pl.BlockSpec((128, 16), lambda i: (0, 0)),
        ],
        out_specs=pl.BlockSpec((BE, 16), lambda i: (i, 0)),
        out_shape=jax.ShapeDtypeStruct((E, 16), jnp.float32),
        interpret=interpret,
    )(prod, ea, Wu)


def kernel(x, edge_index, edge_attr, W1, b1, W2, b2, We1, be1, We2, be2,
           Wc, bc, Wu):
    src = edge_index[0]
    dst = edge_index[1]
    node_m1 = _node_mlp(x, W1, b1, W2, b2)
    g1 = _sc_gather(node_m1, src)
    m = _edge_msg(g1, edge_attr, We1, be1, We2, be2, Wc, bc)
    zeros = jnp.zeros((_NPAD // _NS, 128), jnp.float32)
    p = _sc_scatter_add(m, dst, zeros)
    h_new = _combine(p[0, :10000], p[1, :10000], x)
    prod = _sc_gather_prod(h_new, src, dst)
    e_new = _edge_update(prod, edge_attr, Wu)
    return (h_new, e_new)


# speedup vs baseline: 1.2237x; 1.0801x over previous
import functools

import jax
import jax.numpy as jnp
from jax import lax
from jax.experimental import pallas as pl
from jax.experimental.pallas import tpu as pltpu
from jax.experimental.pallas import tpu_sc as plsc

_NC = 2
_NS = 16
_NW = _NC * _NS

_C = 128
_NPAD = 10240


def _node_mlp(x, W1, b1, W2, b2, interpret=False):
    N = x.shape[0]
    BN = 1000

    def body(x_ref, W1_ref, b1_ref, W2_ref, b2_ref, o_ref):
        h = jnp.maximum(
            jnp.dot(x_ref[...], W1_ref[...], preferred_element_type=jnp.float32)
            + b1_ref[...], 0.0)
        o_ref[...] = (
            jnp.dot(h, W2_ref[...], preferred_element_type=jnp.float32)
            + b2_ref[...])

    return pl.pallas_call(
        body,
        grid=(N // BN,),
        in_specs=[
            pl.BlockSpec((BN, 128), lambda i: (i, 0)),
            pl.BlockSpec((128, 128), lambda i: (0, 0)),
            pl.BlockSpec((1, 128), lambda i: (0, 0)),
            pl.BlockSpec((128, 128), lambda i: (0, 0)),
            pl.BlockSpec((1, 128), lambda i: (0, 0)),
        ],
        out_specs=pl.BlockSpec((BN, 128), lambda i: (i, 0)),
        out_shape=jax.ShapeDtypeStruct((N, 128), jnp.float32),
        interpret=interpret,
    )(x, W1, b1.reshape(1, 128), W2, b2.reshape(1, 128))


def _sc_gather(table, idx):
    D = table.shape[1]
    NT = table.shape[0]
    E = idx.shape[0]
    per_w = E // _NW
    full = per_w // _C
    rem = per_w - full * _C
    NBUF = 2
    LOOK = 1
    ngrp = full // NBUF
    npt = 640
    ntail = NT - 15 * npt
    assert ngrp * NBUF == full and per_w % 8 == 0 and 0 < ntail <= npt
    mesh = plsc.VectorSubcoreMesh(core_axis_name="c", subcore_axis_name="s")

    @functools.partial(
        pl.kernel,
        out_type=jax.ShapeDtypeStruct((E, D), jnp.float32),
        mesh=mesh,
        scratch_types=(
            [pltpu.VMEM((per_w,), jnp.int32),
             pltpu.VMEM((NBUF, _C, D), jnp.float32),
             pltpu.VMEM((rem,), jnp.int32),
             pltpu.VMEM((rem, D), jnp.float32),
             pltpu.VMEM_SHARED((NT, D), jnp.float32)]
            + [pltpu.SemaphoreType.DMA] * (2 * NBUF)
        ),
    )
    def k(table_hbm, idx_hbm, out_hbm, idx_v, rows_v, idx_r, rows_r,
          table_sh, *sems):
        gsem = sems[:NBUF]
        wsem = sems[NBUF:]
        sid = lax.axis_index("s")
        wid = sid * _NC + lax.axis_index("c")
        base = wid * per_w
        @pl.when(sid < 15)
        def _():
            pltpu.sync_copy(table_hbm.at[pl.ds(sid * npt, npt)],
                            table_sh.at[pl.ds(sid * npt, npt)])

        @pl.when(sid == 15)
        def _():
            pltpu.sync_copy(table_hbm.at[pl.ds(15 * npt, ntail)],
                            table_sh.at[pl.ds(15 * npt, ntail)])

        pltpu.sync_copy(idx_hbm.at[pl.ds(base, per_w)], idx_v)
        plsc.subcore_barrier()
        for b in range(LOOK):
            pltpu.async_copy(
                table_sh.at[idx_v.at[pl.ds(b * _C, _C)]],
                rows_v.at[b], gsem[b])

        def outer(g, _):
            for b in range(NBUF):
                c = g * NBUF + b
                b2 = (b + LOOK) % NBUF

                @pl.when(c + LOOK - NBUF >= 0)
                def _():
                    pltpu.make_async_copy(
                        rows_v.at[b2], out_hbm.at[pl.ds(0, _C)],
                        wsem[b2]).wait()

                @pl.when(c + LOOK < full)
                def _():
                    pltpu.async_copy(
                        table_sh.at[idx_v.at[pl.ds((c + LOOK) * _C, _C)]],
                        rows_v.at[b2], gsem[b2])

                pltpu.make_async_copy(
                    table_hbm.at[pl.ds(0, _C)], rows_v.at[b],
                    gsem[b]).wait()
                pltpu.async_copy(rows_v.at[b],
                                 out_hbm.at[pl.ds(base + c * _C, _C)],
                                 wsem[b])
            return ()

        lax.fori_loop(0, ngrp, outer, ())
        pltpu.make_async_copy(rows_v.at[(full - 1) % NBUF],
                              out_hbm.at[pl.ds(0, _C)],
                              wsem[(full - 1) % NBUF]).wait()
        off = base + full * _C
        pltpu.sync_copy(idx_hbm.at[pl.ds(off, rem)], idx_r)
        pltpu.async_copy(table_sh.at[idx_r], rows_r, gsem[0]).wait()
        pltpu.sync_copy(rows_r, out_hbm.at[pl.ds(off, rem)])

    return k(table, idx)


def _sc_gather_pair(table, idx_a, idx_b):
    D = table.shape[1]
    E = idx_a.shape[0]
    per_w = E // _NW
    full = per_w // _C
    rem = per_w - full * _C
    NBUF = 3
    LOOK = 2
    ngrp = full // NBUF
    assert ngrp * NBUF == full and per_w % 8 == 0
    mesh = plsc.VectorSubcoreMesh(core_axis_name="c", subcore_axis_name="s")

    @functools.partial(
        pl.kernel,
        out_type=(jax.ShapeDtypeStruct((E, D), jnp.float32),
                  jax.ShapeDtypeStruct((E, D), jnp.float32)),
        mesh=mesh,
        scratch_types=(
            [pltpu.VMEM((per_w,), jnp.int32),
             pltpu.VMEM((per_w,), jnp.int32),
             pltpu.VMEM((NBUF, _C, D), jnp.float32),
             pltpu.VMEM((NBUF, _C, D), jnp.float32),
             pltpu.VMEM((rem,), jnp.int32),
             pltpu.VMEM((rem, D), jnp.float32)]
            + [pltpu.SemaphoreType.DMA] * (4 * NBUF)
        ),
    )
    def k(table_hbm, ia_hbm, ib_hbm, oa_hbm, ob_hbm, ia_v, ib_v, ra_v, rb_v,
          idx_r, rows_r, *sems):
        gsa = sems[:NBUF]
        gsb = sems[NBUF:2 * NBUF]
        wsa = sems[2 * NBUF:3 * NBUF]
        wsb = sems[3 * NBUF:]
        wid = lax.axis_index("s") * _NC + lax.axis_index("c")
        base = wid * per_w
        pltpu.sync_copy(ia_hbm.at[pl.ds(base, per_w)], ia_v)
        pltpu.sync_copy(ib_hbm.at[pl.ds(base, per_w)], ib_v)
        for b in range(LOOK):
            pltpu.async_copy(table_hbm.at[ia_v.at[pl.ds(b * _C, _C)]],
                             ra_v.at[b], gsa[b])
            pltpu.async_copy(table_hbm.at[ib_v.at[pl.ds(b * _C, _C)]],
                             rb_v.at[b], gsb[b])

        def outer(g, _):
            for b in range(NBUF):
                c = g * NBUF + b
                b2 = (b + LOOK) % NBUF
                for (rv, gs, ws, oh, iv) in ((ra_v, gsa, wsa, oa_hbm, ia_v),
                                             (rb_v, gsb, wsb, ob_hbm, ib_v)):

                    @pl.when(c + LOOK - NBUF >= 0)
                    def _():
                        pltpu.make_async_copy(
                            rv.at[b2], oh.at[pl.ds(0, _C)], ws[b2]).wait()

                    @pl.when(c + LOOK < full)
                    def _():
                        pltpu.async_copy(
                            table_hbm.at[iv.at[pl.ds((c + LOOK) * _C, _C)]],
                            rv.at[b2], gs[b2])

                    pltpu.make_async_copy(
                        table_hbm.at[pl.ds(0, _C)], rv.at[b], gs[b]).wait()
                    pltpu.async_copy(rv.at[b],
                                     oh.at[pl.ds(base + c * _C, _C)], ws[b])
            return ()

        lax.fori_loop(0, ngrp, outer, ())
        for (rv, ws, oh) in ((ra_v, wsa, oa_hbm), (rb_v, wsb, ob_hbm)):
            pltpu.make_async_copy(rv.at[(full - 1) % NBUF],
                                  oh.at[pl.ds(0, _C)],
                                  ws[(full - 1) % NBUF]).wait()
        off = base + full * _C
        for (ih, oh) in ((ia_hbm, oa_hbm), (ib_hbm, ob_hbm)):
            pltpu.sync_copy(ih.at[pl.ds(off, rem)], idx_r)
            pltpu.async_copy(table_hbm.at[idx_r], rows_r, gsa[0]).wait()
            pltpu.sync_copy(rows_r, oh.at[pl.ds(off, rem)])

    return k(table, idx_a, idx_b)


def _sc_gather_prod(table, idx_a, idx_b):
    D = table.shape[1]
    E = idx_a.shape[0]
    per_w = E // _NW
    full = per_w // _C
    rem = per_w - full * _C
    NBUF = 3
    LOOK = 2
    ngrp = full // NBUF
    assert ngrp * NBUF == full and per_w % 8 == 0 and D % 16 == 0
    mesh = plsc.VectorSubcoreMesh(core_axis_name="c", subcore_axis_name="s")

    @functools.partial(
        pl.kernel,
        out_type=jax.ShapeDtypeStruct((E, D), jnp.float32),
        mesh=mesh,
        scratch_types=(
            [pltpu.VMEM((per_w,), jnp.int32),
             pltpu.VMEM((per_w,), jnp.int32),
             pltpu.VMEM((NBUF, _C, D), jnp.float32),
             pltpu.VMEM((NBUF, _C, D), jnp.float32),
             pltpu.VMEM((rem,), jnp.int32),
             pltpu.VMEM((rem, D), jnp.float32),
             pltpu.VMEM((rem, D), jnp.float32)]
            + [pltpu.SemaphoreType.DMA] * (3 * NBUF)
        ),
    )
    def k(table_hbm, ia_hbm, ib_hbm, o_hbm, ia_v, ib_v, ra_v, rb_v,
          idx_r, rowsa_r, rowsb_r, *sems):
        gsa = sems[:NBUF]
        gsb = sems[NBUF:2 * NBUF]
        wsa = sems[2 * NBUF:]
        wid = lax.axis_index("s") * _NC + lax.axis_index("c")
        base = wid * per_w
        pltpu.sync_copy(ia_hbm.at[pl.ds(base, per_w)], ia_v)
        pltpu.sync_copy(ib_hbm.at[pl.ds(base, per_w)], ib_v)
        for b in range(LOOK):
            pltpu.async_copy(table_hbm.at[ia_v.at[pl.ds(b * _C, _C)]],
                             ra_v.at[b], gsa[b])
            pltpu.async_copy(table_hbm.at[ib_v.at[pl.ds(b * _C, _C)]],
                             rb_v.at[b], gsb[b])

        def outer(g, _):
            for b in range(NBUF):
                c = g * NBUF + b
                b2 = (b + LOOK) % NBUF

                @pl.when(c + LOOK - NBUF >= 0)
                def _():
                    pltpu.make_async_copy(
                        ra_v.at[b2], o_hbm.at[pl.ds(0, _C)], wsa[b2]).wait()

                @pl.when(c + LOOK < full)
                def _():
                    pltpu.async_copy(
                        table_hbm.at[ia_v.at[pl.ds((c + LOOK) * _C, _C)]],
                        ra_v.at[b2], gsa[b2])
                    pltpu.async_copy(
                        table_hbm.at[ib_v.at[pl.ds((c + LOOK) * _C, _C)]],
                        rb_v.at[b2], gsb[b2])

                pltpu.make_async_copy(
                    table_hbm.at[pl.ds(0, _C)], ra_v.at[b], gsa[b]).wait()
                pltpu.make_async_copy(
                    table_hbm.at[pl.ds(0, _C)], rb_v.at[b], gsb[b]).wait()

                def mul_row(i, _):
                    for j in range(D // 16):
                        ra_v[b, i, pl.ds(16 * j, 16)] = (
                            ra_v[b, i, pl.ds(16 * j, 16)]
                            * rb_v[b, i, pl.ds(16 * j, 16)])
                    return ()

                lax.fori_loop(0, _C, mul_row, ())
                pltpu.async_copy(ra_v.at[b],
                                 o_hbm.at[pl.ds(base + c * _C, _C)], wsa[b])
            return ()

        lax.fori_loop(0, ngrp, outer, ())
        pltpu.make_async_copy(ra_v.at[(full - 1) % NBUF],
                              o_hbm.at[pl.ds(0, _C)],
                              wsa[(full - 1) % NBUF]).wait()
        off = base + full * _C
        pltpu.sync_copy(ia_hbm.at[pl.ds(off, rem)], idx_r)
        pltpu.async_copy(table_hbm.at[idx_r], rowsa_r, gsa[0]).wait()
        pltpu.sync_copy(ib_hbm.at[pl.ds(off, rem)], idx_r)
        pltpu.async_copy(table_hbm.at[idx_r], rowsb_r, gsb[0]).wait()
        for i in range(rem):
            for j in range(D // 16):
                rowsa_r[i, pl.ds(16 * j, 16)] = (
                    rowsa_r[i, pl.ds(16 * j, 16)]
                    * rowsb_r[i, pl.ds(16 * j, 16)])
        pltpu.sync_copy(rowsa_r, o_hbm.at[pl.ds(off, rem)])

    return k(table, idx_a, idx_b)


def _sc_scatter_add(m, dst, zeros):
    E = m.shape[0]
    per_w = E // _NW
    full = per_w // _C
    rem = per_w - full * _C
    NBUF = 2
    ngrp = full // NBUF
    tail = full - ngrp * NBUF
    mesh = plsc.VectorSubcoreMesh(core_axis_name="c", subcore_axis_name="s")
    rpt = _NPAD // _NS

    @functools.partial(
        pl.kernel,
        out_type=jax.ShapeDtypeStruct((_NC, _NPAD, 128), jnp.float32),
        mesh=mesh,
        scratch_types=(
            [pltpu.VMEM((NBUF, _C), jnp.int32),
             pltpu.VMEM((NBUF, _C, 128), jnp.float32),
             pltpu.VMEM((rem,), jnp.int32),
             pltpu.VMEM((rem, 128), jnp.float32),
             pltpu.VMEM_SHARED((_NPAD, 128), jnp.float32)]
            + [pltpu.SemaphoreType.DMA] * (3 * NBUF)
        ),
    )
    def k(m_hbm, dst_hbm, z_hbm, out_hbm, idx_v, rows_v, idx_r, rows_r,
          agg_sh, *sems):
        isem = sems[:NBUF]
        lsem = sems[NBUF:2 * NBUF]
        ssem = sems[2 * NBUF:]
        cid = lax.axis_index("c")
        sid = lax.axis_index("s")
        wid = sid * _NC + cid
        base = wid * per_w
        pltpu.sync_copy(z_hbm, agg_sh.at[pl.ds(sid * rpt, rpt)])
        plsc.subcore_barrier()
        for b in range(NBUF):
            off = base + b * _C
            pltpu.async_copy(dst_hbm.at[pl.ds(off, _C)], idx_v.at[b],
                             isem[b])
            pltpu.async_copy(m_hbm.at[pl.ds(off, _C)], rows_v.at[b],
                             lsem[b])

        def outer(g, _):
            for b in range(NBUF):
                c = g * NBUF + b
                pltpu.make_async_copy(dst_hbm.at[pl.ds(0, _C)],
                                      idx_v.at[b], isem[b]).wait()
                pltpu.make_async_copy(m_hbm.at[pl.ds(0, _C)],
                                      rows_v.at[b], lsem[b]).wait()
                pltpu.async_copy(rows_v.at[b], agg_sh.at[idx_v.at[b]],
                                 ssem[b], add=True)
                pltpu.make_async_copy(rows_v.at[b], agg_sh.at[idx_v.at[b]],
                                      ssem[b]).wait()

                @pl.when(c + NBUF < full)
                def _():
                    off = base + (c + NBUF) * _C
                    pltpu.async_copy(dst_hbm.at[pl.ds(off, _C)],
                                     idx_v.at[b], isem[b])
                    pltpu.async_copy(m_hbm.at[pl.ds(off, _C)],
                                     rows_v.at[b], lsem[b])
            return ()

        lax.fori_loop(0, ngrp, outer, ())
        for t in range(tail):
            c = ngrp * NBUF + t
            b = c % NBUF
            pltpu.make_async_copy(dst_hbm.at[pl.ds(0, _C)],
                                  idx_v.at[b], isem[b]).wait()
            pltpu.make_async_copy(m_hbm.at[pl.ds(0, _C)],
                                  rows_v.at[b], lsem[b]).wait()
            pltpu.sync_copy(rows_v.at[b], agg_sh.at[idx_v.at[b]], add=True)
        off = base + full * _C
        pltpu.sync_copy(dst_hbm.at[pl.ds(off, rem)], idx_r)
        pltpu.sync_copy(m_hbm.at[pl.ds(off, rem)], rows_r)
        pltpu.sync_copy(rows_r, agg_sh.at[idx_r], add=True)
        plsc.subcore_barrier()
        pltpu.sync_copy(agg_sh.at[pl.ds(sid * rpt, rpt)],
                        out_hbm.at[cid, pl.ds(sid * rpt, rpt)])

    return k(m, dst, zeros)


def _edge_msg(g1, ea, We1, be1, We2, be2, Wc, bc, interpret=False):
    E = g1.shape[0]
    BE = 4000

    def body(g1_ref, ea_ref, We1_ref, be1_ref, We2_ref, be2_ref, Wc_ref,
             bc_ref, o_ref):
        h = jnp.maximum(
            jnp.dot(ea_ref[...], We1_ref[...], preferred_element_type=jnp.float32)
            + be1_ref[...], 0.0)
        m2 = (jnp.dot(h, We2_ref[...], preferred_element_type=jnp.float32)
              + be2_ref[...])
        t = g1_ref[...] * m2
        o_ref[...] = jnp.tanh(
            jnp.dot(t, Wc_ref[...], preferred_element_type=jnp.float32)
            + bc_ref[...])

    return pl.pallas_call(
        body,
        grid=(E // BE,),
        in_specs=[
            pl.BlockSpec((BE, 128), lambda i: (i, 0)),
            pl.BlockSpec((BE, 16), lambda i: (i, 0)),
            pl.BlockSpec((16, 128), lambda i: (0, 0)),
            pl.BlockSpec((1, 128), lambda i: (0, 0)),
            pl.BlockSpec((128, 128), lambda i: (0, 0)),
            pl.BlockSpec((1, 128), lambda i: (0, 0)),
            pl.BlockSpec((128, 128), lambda i: (0, 0)),
            pl.BlockSpec((1, 128), lambda i: (0, 0)),
        ],
        out_specs=pl.BlockSpec((BE, 128), lambda i: (i, 0)),
        out_shape=jax.ShapeDtypeStruct((E, 128), jnp.float32),
        interpret=interpret,
    )(g1, ea, We1, be1.reshape(1, 128), We2, be2.reshape(1, 128), Wc,
      bc.reshape(1, 128))


def _combine(p0, p1, x, interpret=False):
    N = x.shape[0]
    BN = 1000

    def body(a_ref, b_ref, x_ref, o_ref):
        o_ref[...] = a_ref[...] + b_ref[...] + x_ref[...]

    return pl.pallas_call(
        body,
        grid=(N // BN,),
        in_specs=[pl.BlockSpec((BN, 128), lambda i: (i, 0))] * 3,
        out_specs=pl.BlockSpec((BN, 128), lambda i: (i, 0)),
        out_shape=jax.ShapeDtypeStruct((N, 128), jnp.float32),
        interpret=interpret,
    )(p0, p1, x)


def _edge_update(prod, ea, Wu, interpret=False):
    E = prod.shape[0]
    BE = 4000

    def body(pr_ref, ea_ref, Wu_ref, o_ref):
        o_ref[...] = 0.8 * ea_ref[...] + 0.2 * jnp.dot(
            pr_ref[...], Wu_ref[...], preferred_element_type=jnp.float32)

    return pl.pallas_call(
        body,
        grid=(E // BE,),
        in_specs=[
            pl.BlockSpec((BE, 128), lambda i: (i, 0)),
            pl.BlockSpec((BE, 16), lambda i: (i, 0)),
            pl.BlockSpec((128, 16), lambda i: (0, 0)),
        ],
        out_specs=pl.BlockSpec((BE, 16), lambda i: (i, 0)),
        out_shape=jax.ShapeDtypeStruct((E, 16), jnp.float32),
        interpret=interpret,
    )(prod, ea, Wu)


def kernel(x, edge_index, edge_attr, W1, b1, W2, b2, We1, be1, We2, be2,
           Wc, bc, Wu):
    src = edge_index[0]
    dst = edge_index[1]
    node_m1 = _node_mlp(x, W1, b1, W2, b2)
    g1 = _sc_gather(node_m1, src)
    m = _edge_msg(g1, edge_attr, We1, be1, We2, be2, Wc, bc)
    zeros = jnp.zeros((_NPAD // _NS, 128), jnp.float32)
    p = _sc_scatter_add(m, dst, zeros)
    h_new = _combine(p[0, :10000], p[1, :10000], x)
    prod = _sc_gather_prod(h_new, src, dst)
    e_new = _edge_update(prod, edge_attr, Wu)
    return (h_new, e_new)
